# baseline (speedup 1.0000x reference)
import functools

import jax
import jax.numpy as jnp
from jax import lax
from jax.experimental import pallas as pl
from jax.experimental.pallas import tpu as pltpu
from jax.experimental.pallas import tpu_sc as plsc

_VOCAB = 100000
_D = 300
_DP = 304
_B = 16384
_NC = 2
_NS = 16
_NW = _NC * _NS
_BPW = _B // _NW
_CHUNK = 128
_NCHUNK = _BPW // _CHUNK

_mesh = plsc.VectorSubcoreMesh(core_axis_name="c", subcore_axis_name="s")


@functools.partial(
    pl.kernel,
    mesh=_mesh,
    compiler_params=pltpu.CompilerParams(use_tc_tiling_on_sc=False),
    out_type=jax.ShapeDtypeStruct((_B, _DP), jnp.float32),
    scratch_types=[
        pltpu.VMEM((_NCHUNK, _CHUNK), jnp.int32),
        pltpu.VMEM((_CHUNK, _DP), jnp.float32),
        pltpu.VMEM((_CHUNK, _DP), jnp.float32),
        pltpu.SemaphoreType.DMA,
        pltpu.SemaphoreType.DMA,
        pltpu.SemaphoreType.DMA,
        pltpu.SemaphoreType.DMA,
    ],
)
def _gather_kernel(table_hbm, idx_hbm, out_hbm, idx_v, rows0, rows1,
                   gsem0, gsem1, osem0, osem1):
    wid = lax.axis_index("s") * _NC + lax.axis_index("c")
    base = wid * _BPW

    pltpu.sync_copy(idx_hbm.at[wid], idx_v)

    bufs = (rows0, rows1)
    gsems = (gsem0, gsem1)
    osems = (osem0, osem1)

    gathers = [None] * _NCHUNK
    outs = [None] * _NCHUNK
    gathers[0] = pltpu.async_copy(table_hbm.at[idx_v.at[0]], bufs[0], gsems[0])
    for c in range(_NCHUNK):
        nxt = c + 1
        if nxt < _NCHUNK:
            if nxt >= 2:
                outs[nxt - 2].wait()
                outs[nxt - 2] = None
            gathers[nxt] = pltpu.async_copy(
                table_hbm.at[idx_v.at[nxt]], bufs[nxt % 2], gsems[nxt % 2])
        gathers[c].wait()
        outs[c] = pltpu.async_copy(
            bufs[c % 2], out_hbm.at[pl.ds(base + c * _CHUNK, _CHUNK)],
            osems[c % 2])
    for c in range(_NCHUNK):
        if outs[c] is not None:
            outs[c].wait()


def kernel(table, indices):
    table_p = jnp.pad(table, ((0, 0), (0, _DP - _D)))
    idx = indices.astype(jnp.int32).reshape(_NW, _NCHUNK, _CHUNK)
    out = _gather_kernel(table_p, idx)
    return out[:, :_D]


# speedup vs baseline: 1.1123x; 1.1123x over previous
import functools

import jax
import jax.numpy as jnp
from jax import lax
from jax.experimental import pallas as pl
from jax.experimental.pallas import tpu as pltpu
from jax.experimental.pallas import tpu_sc as plsc

_V = 100000
_D = 300
_DP = 384
_B = 16384
_NC = 2
_NS = 16
_NW = _NC * _NS
_BPW = _B // _NW
_CHUNK = 128
_NCHUNK = _BPW // _CHUNK
_NSTRIP = _V // 128
_TAIL = _V - _NSTRIP * 128
_SPW = (_NSTRIP + _NW - 1) // _NW

_mesh = plsc.VectorSubcoreMesh(core_axis_name="c", subcore_axis_name="s")


@functools.partial(
    pl.kernel,
    mesh=_mesh,
    compiler_params=pltpu.CompilerParams(needs_layout_passes=False),
    out_type=jax.ShapeDtypeStruct((_V, _DP), jnp.float32),
    scratch_types=[
        pltpu.VMEM((_D, 128), jnp.float32),
        pltpu.VMEM((128, _DP), jnp.float32),
        pltpu.VMEM((_TAIL, _DP), jnp.float32),
    ],
)
def _transpose_kernel(tt_hbm, tail_hbm, t2_hbm, inb, outb, tailb):
    wid = lax.axis_index("s") * _NC + lax.axis_index("c")
    lanes = lax.iota(jnp.int32, 16)

    @pl.when(wid == 0)
    def _():
        pltpu.sync_copy(tail_hbm, tailb)
        pltpu.sync_copy(tailb, t2_hbm.at[pl.ds(_NSTRIP * 128, _TAIL)])

    def do_strip(ct):
        pltpu.sync_copy(tt_hbm.at[:, pl.ds(ct * 128, 128)], inb)

        def body(r, _):
            col = jnp.full((16,), r, jnp.int32)
            for g in range(8):
                vals = inb[r, pl.ds(g * 16, 16)]
                plsc.store_scatter(outb, [g * 16 + lanes, col], vals)
            return 0

        lax.fori_loop(0, _D, body, 0)
        pltpu.sync_copy(outb, t2_hbm.at[pl.ds(ct * 128, 128)])

    for k in range(_SPW):
        ct = k * _NW + wid

        @pl.when(ct < _NSTRIP)
        def _():
            do_strip(ct)


@functools.partial(
    pl.kernel,
    mesh=_mesh,
    out_type=jax.ShapeDtypeStruct((_B, _DP), jnp.float32),
    scratch_types=[
        pltpu.VMEM((_NCHUNK, _CHUNK), jnp.int32),
        pltpu.VMEM((_CHUNK, _DP), jnp.float32),
        pltpu.VMEM((_CHUNK, _DP), jnp.float32),
        pltpu.SemaphoreType.DMA,
        pltpu.SemaphoreType.DMA,
        pltpu.SemaphoreType.DMA,
        pltpu.SemaphoreType.DMA,
    ],
)
def _gather_kernel(t2_hbm, idx_hbm, out_hbm, idx_v, rows0, rows1,
                   gsem0, gsem1, osem0, osem1):
    wid = lax.axis_index("s") * _NC + lax.axis_index("c")
    base = wid * _BPW

    pltpu.sync_copy(idx_hbm.at[wid], idx_v)

    bufs = (rows0, rows1)
    gsems = (gsem0, gsem1)
    osems = (osem0, osem1)

    def start_gather(c):
        cps = []
        for t in range(3):
            cps.append(pltpu.async_copy(
                t2_hbm.at[idx_v.at[c], pl.ds(t * 128, 128)],
                bufs[c % 2].at[:, pl.ds(t * 128, 128)], gsems[c % 2]))
        return cps

    gathers = [None] * _NCHUNK
    outs = [None] * _NCHUNK
    gathers[0] = start_gather(0)
    for c in range(_NCHUNK):
        nxt = c + 1
        if nxt < _NCHUNK:
            if nxt >= 2:
                outs[nxt - 2].wait()
                outs[nxt - 2] = None
            gathers[nxt] = start_gather(nxt)
        for cp in gathers[c]:
            cp.wait()
        outs[c] = pltpu.async_copy(
            bufs[c % 2], out_hbm.at[pl.ds(base + c * _CHUNK, _CHUNK)],
            osems[c % 2])
    for c in range(_NCHUNK):
        if outs[c] is not None:
            outs[c].wait()


def kernel(table, indices):
    tt = table.T
    tail = jnp.pad(table[_NSTRIP * 128:, :], ((0, 0), (0, _DP - _D)))
    idx = indices.astype(jnp.int32).reshape(_NW, _NCHUNK, _CHUNK)
    t2 = _transpose_kernel(tt, tail)
    out = _gather_kernel(t2, idx)
    return out[:, :_D]


# speedup vs baseline: 1.7140x; 1.5409x over previous
import functools

import jax
import jax.numpy as jnp
from jax import lax
from jax.experimental import pallas as pl
from jax.experimental.pallas import tpu as pltpu
from jax.experimental.pallas import tpu_sc as plsc

_V = 100000
_D = 300
_DP = 384
_B = 16384
_NC = 2
_NS = 16
_NW = _NC * _NS
_BPW = _B // _NW
_CHUNK = 128
_NCHUNK = _BPW // _CHUNK
_NSTRIP = _V // 128
_TAIL = _V - _NSTRIP * 128
_SPW = (_NSTRIP + _NW - 1) // _NW

_mesh = plsc.VectorSubcoreMesh(core_axis_name="c", subcore_axis_name="s")


@functools.partial(
    pl.kernel,
    mesh=_mesh,
    compiler_params=pltpu.CompilerParams(needs_layout_passes=False),
    out_type=jax.ShapeDtypeStruct((_V, _DP), jnp.float32),
    scratch_types=[
        pltpu.VMEM((_D, 128), jnp.float32),
        pltpu.VMEM((_D, 128), jnp.float32),
        pltpu.VMEM((128, 128), jnp.float32),
        pltpu.VMEM((128, 128), jnp.float32),
        pltpu.VMEM((128, 128), jnp.float32),
        pltpu.SemaphoreType.DMA,
        pltpu.SemaphoreType.DMA,
        pltpu.SemaphoreType.DMA,
        pltpu.SemaphoreType.DMA,
        pltpu.SemaphoreType.DMA,
    ],
)
def _transpose_kernel(tt_hbm, tail_hbm, t2_hbm, inb0, inb1, ob0, ob1, ob2,
                      isem0, isem1, osem0, osem1, osem2):
    wid = lax.axis_index("s") * _NC + lax.axis_index("c")
    lanes = lax.iota(jnp.int32, 16)
    inbs = (inb0, inb1)
    isems = (isem0, isem1)
    obs = (ob0, ob1, ob2)
    osems = (osem0, osem1, osem2)

    def strip(k):
        ct = jnp.minimum(k * _NW + wid, _NSTRIP - 1)
        return pl.multiple_of(ct * 128, 128)

    def start_in(k):
        return pltpu.async_copy(
            tt_hbm.at[:, pl.ds(strip(k), 128)], inbs[k % 2], isems[k % 2])

    ins = [None] * _SPW
    outs = [[None] * 3 for _ in range(_SPW)]
    ins[0] = start_in(0)

    for k in range(_SPW):
        row0 = strip(k)
        ins[k].wait()
        if k + 1 < _SPW:
            ins[k + 1] = start_in(k + 1)
        inb = inbs[k % 2]
        for t in range(3):
            lo = t * 128
            hi = min((t + 1) * 128, _D)
            if k > 0:
                outs[k - 1][t].wait()
                outs[k - 1][t] = None

            @plsc.parallel_loop(lo, hi, unroll=2)
            def _(r):
                col = jnp.full((16,), r - lo, jnp.int32)
                for g in range(8):
                    vals = inb[r, pl.ds(g * 16, 16)]
                    plsc.store_scatter(obs[t], [g * 16 + lanes, col], vals)

            outs[k][t] = pltpu.async_copy(
                obs[t],
                t2_hbm.at[pl.ds(row0, 128), pl.ds(t * 128, 128)],
                osems[t])

    for t in range(3):
        outs[_SPW - 1][t].wait()

    @pl.when(wid == 0)
    def _():
        for t in range(3):
            pltpu.sync_copy(tail_hbm.at[:, pl.ds(t * 128, 128)],
                            obs[t].at[pl.ds(0, _TAIL)])
            pltpu.sync_copy(obs[t].at[pl.ds(0, _TAIL)],
                            t2_hbm.at[pl.ds(_NSTRIP * 128, _TAIL),
                                      pl.ds(t * 128, 128)])


@functools.partial(
    pl.kernel,
    mesh=_mesh,
    out_type=jax.ShapeDtypeStruct((_B, _DP), jnp.float32),
    scratch_types=[
        pltpu.VMEM((_NCHUNK, _CHUNK), jnp.int32),
        pltpu.VMEM((_CHUNK, _DP), jnp.float32),
        pltpu.VMEM((_CHUNK, _DP), jnp.float32),
        pltpu.SemaphoreType.DMA,
        pltpu.SemaphoreType.DMA,
        pltpu.SemaphoreType.DMA,
        pltpu.SemaphoreType.DMA,
    ],
)
def _gather_kernel(t2_hbm, idx_hbm, out_hbm, idx_v, rows0, rows1,
                   gsem0, gsem1, osem0, osem1):
    wid = lax.axis_index("s") * _NC + lax.axis_index("c")
    base = wid * _BPW

    pltpu.sync_copy(idx_hbm.at[wid], idx_v)

    bufs = (rows0, rows1)
    gsems = (gsem0, gsem1)
    osems = (osem0, osem1)

    def start_gather(c):
        cps = []
        for t in range(3):
            cps.append(pltpu.async_copy(
                t2_hbm.at[idx_v.at[c], pl.ds(t * 128, 128)],
                bufs[c % 2].at[:, pl.ds(t * 128, 128)], gsems[c % 2]))
        return cps

    gathers = [None] * _NCHUNK
    outs = [None] * _NCHUNK
    gathers[0] = start_gather(0)
    for c in range(_NCHUNK):
        nxt = c + 1
        if nxt < _NCHUNK:
            if nxt >= 2:
                outs[nxt - 2].wait()
                outs[nxt - 2] = None
            gathers[nxt] = start_gather(nxt)
        for cp in gathers[c]:
            cp.wait()
        outs[c] = pltpu.async_copy(
            bufs[c % 2], out_hbm.at[pl.ds(base + c * _CHUNK, _CHUNK)],
            osems[c % 2])
    for c in range(_NCHUNK):
        if outs[c] is not None:
            outs[c].wait()


def kernel(table, indices):
    tt = table.T
    tail = jnp.pad(table[_NSTRIP * 128:, :], ((0, 0), (0, _DP - _D)))
    idx = indices.astype(jnp.int32).reshape(_NW, _NCHUNK, _CHUNK)
    t2 = _transpose_kernel(tt, tail)
    out = _gather_kernel(t2, idx)
    return out[:, :_D]
